# baseline (speedup 1.0000x reference)
import functools

import jax
import jax.numpy as jnp
from jax import lax
from jax.experimental import pallas as pl
from jax.experimental.pallas import tpu as pltpu
from jax.experimental.pallas import tpu_sc as plsc

_TOPK = 5
_NC, _NS = 2, 16
_LANES = 16


def _lstm_body(ts_ref, wih_ref, whh_ref, bias_ref, wqt_ref, q_ref):
    L, B, F = ts_ref.shape
    H = whh_ref.shape[0]

    def sig(x):
        return 1.0 / (1.0 + jnp.exp(-x))

    def step(t, carry):
        h, c = carry
        xt = ts_ref[pl.ds(t, 1), :, :].reshape(B, F)
        gates = (jnp.dot(xt, wih_ref[...], preferred_element_type=jnp.float32)
                 + jnp.dot(h, whh_ref[...], preferred_element_type=jnp.float32)
                 + bias_ref[...])
        i_g = gates[:, 0 * H:1 * H]
        f_g = gates[:, 1 * H:2 * H]
        g_g = gates[:, 2 * H:3 * H]
        o_g = gates[:, 3 * H:4 * H]
        c2 = sig(f_g) * c + sig(i_g) * jnp.tanh(g_g)
        h2 = sig(o_g) * jnp.tanh(c2)
        return (h2, c2)

    init = (jnp.zeros((B, H), jnp.float32), jnp.zeros((B, H), jnp.float32))
    h, _ = lax.fori_loop(0, L, step, init)
    q_ref[...] = jnp.dot(h, wqt_ref[...], preferred_element_type=jnp.float32)


def _attn_body(inv_scale, raw_ref, w_ref, q_ref, out_ref):
    mL, F, nL = raw_ref.shape[1], raw_ref.shape[2], raw_ref.shape[3]
    slab = raw_ref[0].reshape(mL * F, nL)
    keys = jnp.dot(w_ref[...], slab, preferred_element_type=jnp.float32)
    out_ref[0] = jnp.dot(q_ref[0, 0], keys,
                         preferred_element_type=jnp.float32) * inv_scale


def _sc_topk_body(n_scores, n_leaders, m_lags, attn_hbm, idx_hbm, sco_hbm,
                  abuf, lbuf, gbuf, sbuf):
    wid = lax.axis_index("c") * _NS + lax.axis_index("s")
    pltpu.sync_copy(attn_hbm.at[wid], abuf)

    lane = lax.iota(jnp.int32, _LANES)
    lane_m = lane * m_lags
    neg = jnp.full((_LANES,), -jnp.inf, jnp.float32)
    zero_i = jnp.zeros((_LANES,), jnp.int32)
    cpl = n_leaders // _LANES

    def step(i, carry):
        cv, ci = carry
        v = abuf[pl.ds(i * _LANES, _LANES)]

        def merge(_):
            t = lax.rem(i, cpl) * (_LANES * m_lags) + lax.div(i, cpl)
            iv = lane_m + t
            nv, ni = plsc.sort_key_val(v, iv, descending=True)
            take = cv >= nv
            tv = jnp.where(take, cv, nv)
            ti = jnp.where(take, ci, ni)
            return tuple(plsc.sort_key_val(tv, ti, descending=False))

        def skip(_):
            return (cv, ci)

        return lax.cond(jnp.any(v > cv[0]), merge, skip, None)

    cv, ci = lax.fori_loop(0, n_scores // _LANES, step, (neg, zero_i))
    dv = lax.rev(cv, (0,))
    di = lax.rev(ci, (0,))

    mask5 = lane < _TOPK
    m = jnp.max(dv)
    e = jnp.where(mask5, jnp.exp(dv - m), 0.0)
    s = jnp.sum(e)
    svec = jnp.full((_LANES,), 1.0, jnp.float32) * s
    sc16 = e / svec

    sbuf[...] = sc16
    lbuf[...] = lax.div(di, m_lags)
    gbuf[...] = lax.rem(di, m_lags)
    pltpu.sync_copy(lbuf, idx_hbm.at[wid, 0])
    pltpu.sync_copy(gbuf, idx_hbm.at[wid, 1])
    pltpu.sync_copy(sbuf, sco_hbm.at[wid])


def _sc_topk_call(attn_lin, N, nL, mL, B):
    mesh = plsc.VectorSubcoreMesh(core_axis_name="c", subcore_axis_name="s",
                                  num_cores=_NC, num_subcores=_NS)
    sc_fn = pl.kernel(
        functools.partial(_sc_topk_body, N, nL, mL),
        out_type=[
            jax.ShapeDtypeStruct((B, 2, _LANES), jnp.int32),
            jax.ShapeDtypeStruct((B, _LANES), jnp.float32),
        ],
        mesh=mesh,
        compiler_params=pltpu.CompilerParams(needs_layout_passes=False,
                                             use_tc_tiling_on_sc=False),
        scratch_types=[
            pltpu.VMEM((N,), jnp.float32),
            pltpu.VMEM((_LANES,), jnp.int32),
            pltpu.VMEM((_LANES,), jnp.int32),
            pltpu.VMEM((_LANES,), jnp.float32),
        ],
    )
    return sc_fn(attn_lin)


def _head_body(sel_ref, sco_ref, r0_ref, r1_ref, r2_ref, r3_ref, r4_ref,
               p1_ref, b1_ref, p2_ref, b2_ref, p3_ref, b3_ref, pred_ref):
    b = pl.program_id(0)
    F = r0_ref.shape[1]
    iot = lax.broadcasted_iota(jnp.int32, (F, 128), 1)
    acc = None
    for k, rref in enumerate((r0_ref, r1_ref, r2_ref, r3_ref, r4_ref)):
        roff = lax.rem(sel_ref[b, 0, k], 128)
        col = jnp.sum(jnp.where(iot == roff, rref[0], 0.0), axis=1,
                      keepdims=True)
        contrib = col * sco_ref[b, k]
        acc = contrib if k == 0 else acc + contrib
    h1 = jnp.maximum(
        jnp.dot(p1_ref[...], acc, preferred_element_type=jnp.float32)
        + b1_ref[...], 0.0)
    h2 = jnp.maximum(
        jnp.dot(p2_ref[...], h1, preferred_element_type=jnp.float32)
        + b2_ref[...], 0.0)
    pred_ref[0] = (jnp.dot(p3_ref[...], h2,
                           preferred_element_type=jnp.float32) + b3_ref[...])


def kernel(target_seq, leader_raw_features, W_ih, W_hh, b_ih, b_hh, W_Q, W_K,
           P1_w, P1_b, P2_w, P2_b, P3_w, P3_b):
    B, L, F = target_seq.shape
    _, nL, mL, _ = leader_raw_features.shape
    H = W_hh.shape[1]
    N = nL * mL
    inv_scale = 1.0 / (H ** 0.5)

    ts_t = jnp.swapaxes(target_seq, 0, 1)
    bias = (b_ih + b_hh).reshape(1, 4 * H)
    q = pl.pallas_call(
        _lstm_body,
        out_shape=jax.ShapeDtypeStruct((B, H), jnp.float32),
    )(ts_t, W_ih.T, W_hh.T, bias, W_Q.T)

    rawT = jnp.transpose(leader_raw_features, (0, 2, 3, 1))
    eye = jnp.eye(mL, dtype=jnp.float32)
    wbd = jnp.kron(eye, W_K)
    qbd = (eye[None, :, :, None] * q[:, None, None, :]).reshape(B, 1, mL, mL * F)
    attnT = pl.pallas_call(
        functools.partial(_attn_body, inv_scale),
        grid=(B,),
        in_specs=[
            pl.BlockSpec((1, mL, F, nL), lambda b: (b, 0, 0, 0)),
            pl.BlockSpec((mL * F, mL * F), lambda b: (0, 0)),
            pl.BlockSpec((1, 1, mL, mL * F), lambda b: (b, 0, 0, 0)),
        ],
        out_specs=pl.BlockSpec((1, mL, nL), lambda b: (b, 0, 0)),
        out_shape=jax.ShapeDtypeStruct((B, mL, nL), jnp.float32),
    )(rawT, wbd, qbd.reshape(B, 1, mL, mL * F))
    attn = jnp.transpose(attnT, (0, 2, 1))

    attn_lin = attnT.reshape(B, N)
    idxo, sco = _sc_topk_call(attn_lin, N, nL, mL, B)

    rawT2 = rawT.reshape(B, mL * F, nL)

    def _col_spec(k):
        return pl.BlockSpec(
            (1, F, 128),
            lambda b, sel, sco, _k=k: (b, sel[b, 1, _k], sel[b, 0, _k] // 128))

    grid_spec = pltpu.PrefetchScalarGridSpec(
        num_scalar_prefetch=2,
        grid=(B,),
        in_specs=[_col_spec(k) for k in range(_TOPK)] + [
            pl.BlockSpec((F, F), lambda b, sel, sco: (0, 0)),
            pl.BlockSpec((F, 1), lambda b, sel, sco: (0, 0)),
            pl.BlockSpec((H // 2, F), lambda b, sel, sco: (0, 0)),
            pl.BlockSpec((H // 2, 1), lambda b, sel, sco: (0, 0)),
            pl.BlockSpec((1, H // 2), lambda b, sel, sco: (0, 0)),
            pl.BlockSpec((1, 1), lambda b, sel, sco: (0, 0)),
        ],
        out_specs=pl.BlockSpec((1, 1, 1), lambda b, sel, sco: (b, 0, 0)),
    )
    pred3 = pl.pallas_call(
        _head_body,
        grid_spec=grid_spec,
        out_shape=jax.ShapeDtypeStruct((B, 1, 1), jnp.float32),
    )(idxo, sco, rawT2, rawT2, rawT2, rawT2, rawT2, P1_w, P1_b.reshape(F, 1),
      P2_w, P2_b.reshape(H // 2, 1), P3_w, P3_b.reshape(1, 1))

    pred = pred3.reshape(B, 1)
    top_k_indices = jnp.stack([idxo[:, 0, :_TOPK], idxo[:, 1, :_TOPK]], axis=-1)
    top_k_scores = sco[:, :_TOPK]
    return (pred, top_k_indices, top_k_scores, attn)


# speedup vs baseline: 1.2518x; 1.2518x over previous
import functools

import jax
import jax.numpy as jnp
from jax import lax
from jax.experimental import pallas as pl
from jax.experimental.pallas import tpu as pltpu
from jax.experimental.pallas import tpu_sc as plsc

_TOPK = 5
_NC, _NS = 2, 16
_LANES = 16


def _lstm_body(ts_ref, wih_ref, whh_ref, bias_ref, wqt_ref, q_ref):
    L, B, F = ts_ref.shape
    H = whh_ref.shape[0]

    def sig(x):
        return 1.0 / (1.0 + jnp.exp(-x))

    def step(t, carry):
        h, c = carry
        xt = ts_ref[pl.ds(t, 1), :, :].reshape(B, F)
        gates = (jnp.dot(xt, wih_ref[...], preferred_element_type=jnp.float32)
                 + jnp.dot(h, whh_ref[...], preferred_element_type=jnp.float32)
                 + bias_ref[...])
        i_g = gates[:, 0 * H:1 * H]
        f_g = gates[:, 1 * H:2 * H]
        g_g = gates[:, 2 * H:3 * H]
        o_g = gates[:, 3 * H:4 * H]
        c2 = sig(f_g) * c + sig(i_g) * jnp.tanh(g_g)
        h2 = sig(o_g) * jnp.tanh(c2)
        return (h2, c2)

    init = (jnp.zeros((B, H), jnp.float32), jnp.zeros((B, H), jnp.float32))
    h, _ = lax.fori_loop(0, L, step, init)
    q_ref[...] = jnp.dot(h, wqt_ref[...], preferred_element_type=jnp.float32)


def _attn_body(inv_scale, raw_ref, w_ref, q_ref, out_ref):
    mL, F, nL = raw_ref.shape[1], raw_ref.shape[2], raw_ref.shape[3]
    slab = raw_ref[0].reshape(mL * F, nL)
    keys = jnp.dot(w_ref[...], slab, preferred_element_type=jnp.float32)
    out_ref[0] = jnp.dot(q_ref[0, 0], keys,
                         preferred_element_type=jnp.float32) * inv_scale


def _sc_topk_body(n_scores, n_leaders, m_lags, attn_hbm, idx_hbm, sco_hbm,
                  abuf, lbuf, gbuf, sbuf):
    wid = lax.axis_index("c") * _NS + lax.axis_index("s")
    pltpu.sync_copy(attn_hbm.at[wid], abuf)

    lane = lax.iota(jnp.int32, _LANES)
    lane_m = lane * m_lags
    neg = jnp.full((_LANES,), -jnp.inf, jnp.float32)
    zero_i = jnp.zeros((_LANES,), jnp.int32)
    cpl = n_leaders // _LANES

    def step(i, carry):
        cv, ci = carry
        v = abuf[pl.ds(i * _LANES, _LANES)]
        t = lax.rem(i, cpl) * (_LANES * m_lags) + lax.div(i, cpl)
        iv = lane_m + t
        nv, ni = plsc.sort_key_val(v, iv, descending=True)
        take = cv >= nv
        tv = jnp.where(take, cv, nv)
        ti = jnp.where(take, ci, ni)
        return tuple(plsc.sort_key_val(tv, ti, descending=False))

    cv, ci = lax.fori_loop(0, n_scores // _LANES, step, (neg, zero_i))
    dv = lax.rev(cv, (0,))
    di = lax.rev(ci, (0,))

    mask5 = lane < _TOPK
    m = jnp.max(dv)
    e = jnp.where(mask5, jnp.exp(dv - m), 0.0)
    s = jnp.sum(e)
    svec = jnp.full((_LANES,), 1.0, jnp.float32) * s
    sc16 = e / svec

    sbuf[...] = sc16
    lbuf[...] = lax.div(di, m_lags)
    gbuf[...] = lax.rem(di, m_lags)
    pltpu.sync_copy(lbuf, idx_hbm.at[wid, 0])
    pltpu.sync_copy(gbuf, idx_hbm.at[wid, 1])
    pltpu.sync_copy(sbuf, sco_hbm.at[wid])


def _sc_topk_call(attn_lin, N, nL, mL, B):
    mesh = plsc.VectorSubcoreMesh(core_axis_name="c", subcore_axis_name="s",
                                  num_cores=_NC, num_subcores=_NS)
    sc_fn = pl.kernel(
        functools.partial(_sc_topk_body, N, nL, mL),
        out_type=[
            jax.ShapeDtypeStruct((B, 2, _LANES), jnp.int32),
            jax.ShapeDtypeStruct((B, _LANES), jnp.float32),
        ],
        mesh=mesh,
        compiler_params=pltpu.CompilerParams(needs_layout_passes=False,
                                             use_tc_tiling_on_sc=False),
        scratch_types=[
            pltpu.VMEM((N,), jnp.float32),
            pltpu.VMEM((_LANES,), jnp.int32),
            pltpu.VMEM((_LANES,), jnp.int32),
            pltpu.VMEM((_LANES,), jnp.float32),
        ],
    )
    return sc_fn(attn_lin)


def _head_body(sel_ref, sco_ref, r0_ref, r1_ref, r2_ref, r3_ref, r4_ref,
               p1_ref, b1_ref, p2_ref, b2_ref, p3_ref, b3_ref, pred_ref):
    b = pl.program_id(0)
    F = r0_ref.shape[1]
    iot = lax.broadcasted_iota(jnp.int32, (F, 128), 1)
    acc = None
    for k, rref in enumerate((r0_ref, r1_ref, r2_ref, r3_ref, r4_ref)):
        roff = lax.rem(sel_ref[b, 0, k], 128)
        col = jnp.sum(jnp.where(iot == roff, rref[0], 0.0), axis=1,
                      keepdims=True)
        contrib = col * sco_ref[b, k]
        acc = contrib if k == 0 else acc + contrib
    h1 = jnp.maximum(
        jnp.dot(p1_ref[...], acc, preferred_element_type=jnp.float32)
        + b1_ref[...], 0.0)
    h2 = jnp.maximum(
        jnp.dot(p2_ref[...], h1, preferred_element_type=jnp.float32)
        + b2_ref[...], 0.0)
    pred_ref[0] = (jnp.dot(p3_ref[...], h2,
                           preferred_element_type=jnp.float32) + b3_ref[...])


def kernel(target_seq, leader_raw_features, W_ih, W_hh, b_ih, b_hh, W_Q, W_K,
           P1_w, P1_b, P2_w, P2_b, P3_w, P3_b):
    B, L, F = target_seq.shape
    _, nL, mL, _ = leader_raw_features.shape
    H = W_hh.shape[1]
    N = nL * mL
    inv_scale = 1.0 / (H ** 0.5)

    ts_t = jnp.swapaxes(target_seq, 0, 1)
    bias = (b_ih + b_hh).reshape(1, 4 * H)
    q = pl.pallas_call(
        _lstm_body,
        out_shape=jax.ShapeDtypeStruct((B, H), jnp.float32),
    )(ts_t, W_ih.T, W_hh.T, bias, W_Q.T)

    rawT = jnp.transpose(leader_raw_features, (0, 2, 3, 1))
    eye = jnp.eye(mL, dtype=jnp.float32)
    wbd = jnp.kron(eye, W_K)
    qbd = (eye[None, :, :, None] * q[:, None, None, :]).reshape(B, 1, mL, mL * F)
    attnT = pl.pallas_call(
        functools.partial(_attn_body, inv_scale),
        grid=(B,),
        in_specs=[
            pl.BlockSpec((1, mL, F, nL), lambda b: (b, 0, 0, 0)),
            pl.BlockSpec((mL * F, mL * F), lambda b: (0, 0)),
            pl.BlockSpec((1, 1, mL, mL * F), lambda b: (b, 0, 0, 0)),
        ],
        out_specs=pl.BlockSpec((1, mL, nL), lambda b: (b, 0, 0)),
        out_shape=jax.ShapeDtypeStruct((B, mL, nL), jnp.float32),
    )(rawT, wbd, qbd.reshape(B, 1, mL, mL * F))
    attn = jnp.transpose(attnT, (0, 2, 1))

    attn_lin = attnT.reshape(B, N)
    idxo, sco = _sc_topk_call(attn_lin, N, nL, mL, B)

    rawT2 = rawT.reshape(B, mL * F, nL)

    def _col_spec(k):
        return pl.BlockSpec(
            (1, F, 128),
            lambda b, sel, sco, _k=k: (b, sel[b, 1, _k], sel[b, 0, _k] // 128))

    grid_spec = pltpu.PrefetchScalarGridSpec(
        num_scalar_prefetch=2,
        grid=(B,),
        in_specs=[_col_spec(k) for k in range(_TOPK)] + [
            pl.BlockSpec((F, F), lambda b, sel, sco: (0, 0)),
            pl.BlockSpec((F, 1), lambda b, sel, sco: (0, 0)),
            pl.BlockSpec((H // 2, F), lambda b, sel, sco: (0, 0)),
            pl.BlockSpec((H // 2, 1), lambda b, sel, sco: (0, 0)),
            pl.BlockSpec((1, H // 2), lambda b, sel, sco: (0, 0)),
            pl.BlockSpec((1, 1), lambda b, sel, sco: (0, 0)),
        ],
        out_specs=pl.BlockSpec((1, 1, 1), lambda b, sel, sco: (b, 0, 0)),
    )
    pred3 = pl.pallas_call(
        _head_body,
        grid_spec=grid_spec,
        out_shape=jax.ShapeDtypeStruct((B, 1, 1), jnp.float32),
    )(idxo, sco, rawT2, rawT2, rawT2, rawT2, rawT2, P1_w, P1_b.reshape(F, 1),
      P2_w, P2_b.reshape(H // 2, 1), P3_w, P3_b.reshape(1, 1))

    pred = pred3.reshape(B, 1)
    top_k_indices = jnp.stack([idxo[:, 0, :_TOPK], idxo[:, 1, :_TOPK]], axis=-1)
    top_k_scores = sco[:, :_TOPK]
    return (pred, top_k_indices, top_k_scores, attn)
